# baseline (speedup 1.0000x reference)
import functools

import numpy as np

import jax
import jax.numpy as jnp
from jax import lax
from jax.experimental import pallas as pl
from jax.experimental.pallas import tpu as pltpu
from jax.experimental.pallas import tpu_sc as plsc

_NC = 2
_NS = 16
_NW = _NC * _NS


def _tc_matmul_pack(x, W_ab, b_ab):
    n, d_in = x.shape
    d_out = W_ab.shape[1]
    dw = d_out // 2

    def body(x_ref, w_ref, b_ref, o_ref):
        y = (
            jnp.dot(x_ref[...], w_ref[...], preferred_element_type=jnp.float32)
            + b_ref[...]
        )
        ybits = lax.bitcast_convert_type(
            y.astype(jnp.bfloat16).astype(jnp.float32), jnp.int32
        )
        a = ybits[:, :dw]
        bb = ybits[:, dw:]
        o_ref[...] = lax.shift_right_logical(a, 16) | (bb & jnp.int32(-0x10000))

    nblk = 10
    blk = n // nblk
    return pl.pallas_call(
        body,
        grid=(nblk,),
        in_specs=[
            pl.BlockSpec((blk, d_in), lambda i: (i, 0)),
            pl.BlockSpec((d_in, d_out), lambda i: (0, 0)),
            pl.BlockSpec((1, d_out), lambda i: (0, 0)),
        ],
        out_specs=pl.BlockSpec((blk, dw), lambda i: (i, 0)),
        out_shape=jax.ShapeDtypeStruct((n, dw), jnp.int32),
    )(x, W_ab, b_ab)


def _make_sc_gather_add(E, D, C, NBUF):
    epw = E // _NW
    nchunks = epw // C
    nouter = nchunks // NBUF
    assert nchunks == nouter * NBUF
    assert NBUF >= 4 and epw % C == 0 and C % 8 == 0 and C <= 128
    Dw = D // 2
    mesh = plsc.VectorSubcoreMesh(core_axis_name="c", subcore_axis_name="s")

    @functools.partial(
        pl.kernel,
        mesh=mesh,
        compiler_params=pltpu.CompilerParams(
            needs_layout_passes=False, use_tc_tiling_on_sc=False),
        out_type=jax.ShapeDtypeStruct((E, D), jnp.float32),
        scratch_types=[
            pltpu.VMEM((epw,), jnp.int32),
            pltpu.VMEM((epw,), jnp.int32),
            pltpu.VMEM((NBUF, C, Dw), jnp.int32),
            pltpu.VMEM((NBUF, C, Dw), jnp.int32),
            pltpu.VMEM((NBUF, C, D), jnp.float32),
            pltpu.SemaphoreType.DMA((NBUF,)),
            pltpu.SemaphoreType.DMA((NBUF,)),
        ],
    )
    def sc_fn(y_hbm, ei_hbm, out_hbm,
              idxr, idxc, bufa, bufb, bufo, gsem, wsem):
        wid = lax.axis_index("s") * _NC + lax.axis_index("c")
        base = wid * epw

        pltpu.sync_copy(ei_hbm.at[0, pl.ds(base, epw)], idxr)
        pltpu.sync_copy(ei_hbm.at[1, pl.ds(base, epw)], idxc)

        def fire_gather(j, s):
            o = j * C
            pltpu.async_copy(y_hbm.at[idxr.at[pl.ds(o, C)]], bufa.at[s], gsem.at[s])
            pltpu.async_copy(y_hbm.at[idxc.at[pl.ds(o, C)]], bufb.at[s], gsem.at[s])

        def wait_gather(j, s):
            o = j * C
            pltpu.make_async_copy(
                y_hbm.at[idxr.at[pl.ds(o, C)]], bufa.at[s], gsem.at[s]).wait()
            pltpu.make_async_copy(
                y_hbm.at[idxc.at[pl.ds(o, C)]], bufb.at[s], gsem.at[s]).wait()

        def fire_write(j, s):
            o = base + j * C
            pltpu.async_copy(bufo.at[s], out_hbm.at[pl.ds(o, C)], wsem.at[s])

        def wait_write(j, s):
            o = base + j * C
            pltpu.make_async_copy(
                bufo.at[s], out_hbm.at[pl.ds(o, C)], wsem.at[s]).wait()

        himask = jnp.full((16,), -0x10000, dtype=jnp.int32)

        def widen_lo(v):
            return plsc.bitcast(v << 16, jnp.float32)

        def widen_hi(v):
            return plsc.bitcast(v & himask, jnp.float32)

        def do_add(s):
            @plsc.parallel_loop(0, C, unroll=4)
            def _add_body(e):
                for g in range(Dw // 16):
                    sl = pl.ds(g * 16, 16)
                    va = bufa[s, e, sl]
                    vb = bufb[s, e, sl]
                    bufo[s, e, pl.ds(g * 32, 16)] = widen_lo(va) + widen_lo(vb)
                    bufo[s, e, pl.ds(g * 32 + 16, 16)] = widen_hi(va) + widen_hi(vb)

        fire_gather(0, 0)
        fire_gather(1, 1)

        def outer(jj, carry):
            for s in range(NBUF):
                j = jj * NBUF + s
                if s < 2:
                    @pl.when(jj >= 1)
                    def _w():
                        wait_write(j - 2, (s - 2) % NBUF)
                else:
                    wait_write(j - 2, s - 2)
                if s < NBUF - 2:
                    fire_gather(j + 2, (s + 2) % NBUF)
                else:
                    @pl.when(jj < nouter - 1)
                    def _g():
                        fire_gather(j + 2, (s + 2) % NBUF)
                wait_gather(j, s)
                do_add(s)
                fire_write(j, s)
            return carry

        lax.fori_loop(0, nouter, outer, 0, unroll=False)

        for t in (nchunks - 2, nchunks - 1):
            wait_write(t, t % NBUF)

    return sc_fn


def kernel(x, edge_index, W, b):
    n, d_in = x.shape
    d_out = W.shape[1]
    E = edge_index.shape[1]

    cols_a = np.array([32 * g + k for g in range(d_out // 32) for k in range(16)])
    perm_ab = np.concatenate([cols_a, cols_a + 16])
    b_ab = (0.5 * b)[perm_ab].reshape(1, d_out).astype(jnp.float32)
    y32 = _tc_matmul_pack(x, W[:, perm_ab], b_ab)

    sc_fn = _make_sc_gather_add(E, d_out, C=80, NBUF=5)
    return sc_fn(y32, edge_index)


# speedup vs baseline: 1.0264x; 1.0264x over previous
import functools

import numpy as np

import jax
import jax.numpy as jnp
from jax import lax
from jax.experimental import pallas as pl
from jax.experimental.pallas import tpu as pltpu
from jax.experimental.pallas import tpu_sc as plsc

_NC = 2
_NS = 16
_NW = _NC * _NS


def _tc_matmul_pack(x, W_ab, b_ab):
    n, d_in = x.shape
    d_out = W_ab.shape[1]
    dw = d_out // 2

    def body(x_ref, w_ref, b_ref, o_ref):
        y = (
            jnp.dot(x_ref[...], w_ref[...], preferred_element_type=jnp.float32)
            + b_ref[...]
        )
        ybits = lax.bitcast_convert_type(
            y.astype(jnp.bfloat16).astype(jnp.float32), jnp.int32
        )
        a = ybits[:, :dw]
        bb = ybits[:, dw:]
        o_ref[...] = lax.shift_right_logical(a, 16) | (bb & jnp.int32(-0x10000))

    return pl.pallas_call(
        body,
        out_shape=jax.ShapeDtypeStruct((n, dw), jnp.int32),
    )(x, W_ab, b_ab)


def _make_sc_gather_add(E, D, C, NBUF):
    epw = E // _NW
    nchunks = epw // C
    nouter = nchunks // NBUF
    assert nchunks == nouter * NBUF
    assert NBUF >= 4 and epw % C == 0 and C % 8 == 0 and C <= 128
    Dw = D // 2
    mesh = plsc.VectorSubcoreMesh(core_axis_name="c", subcore_axis_name="s")

    @functools.partial(
        pl.kernel,
        mesh=mesh,
        compiler_params=pltpu.CompilerParams(
            needs_layout_passes=False, use_tc_tiling_on_sc=False),
        out_type=jax.ShapeDtypeStruct((E, D), jnp.float32),
        scratch_types=[
            pltpu.VMEM((epw,), jnp.int32),
            pltpu.VMEM((epw,), jnp.int32),
            pltpu.VMEM((NBUF, C, Dw), jnp.int32),
            pltpu.VMEM((NBUF, C, Dw), jnp.int32),
            pltpu.VMEM((NBUF, C, D), jnp.float32),
            pltpu.SemaphoreType.DMA((NBUF,)),
            pltpu.SemaphoreType.DMA((NBUF,)),
        ],
    )
    def sc_fn(y_hbm, ei_hbm, out_hbm,
              idxr, idxc, bufa, bufb, bufo, gsem, wsem):
        wid = lax.axis_index("s") * _NC + lax.axis_index("c")
        base = wid * epw

        pltpu.sync_copy(ei_hbm.at[0, pl.ds(base, epw)], idxr)
        pltpu.sync_copy(ei_hbm.at[1, pl.ds(base, epw)], idxc)

        def fire_gather(j, s):
            o = j * C
            pltpu.async_copy(y_hbm.at[idxr.at[pl.ds(o, C)]], bufa.at[s], gsem.at[s])
            pltpu.async_copy(y_hbm.at[idxc.at[pl.ds(o, C)]], bufb.at[s], gsem.at[s])

        def wait_gather(j, s):
            o = j * C
            pltpu.make_async_copy(
                y_hbm.at[idxr.at[pl.ds(o, C)]], bufa.at[s], gsem.at[s]).wait()
            pltpu.make_async_copy(
                y_hbm.at[idxc.at[pl.ds(o, C)]], bufb.at[s], gsem.at[s]).wait()

        def fire_write(j, s):
            o = base + j * C
            pltpu.async_copy(bufo.at[s], out_hbm.at[pl.ds(o, C)], wsem.at[s])

        def wait_write(j, s):
            o = base + j * C
            pltpu.make_async_copy(
                bufo.at[s], out_hbm.at[pl.ds(o, C)], wsem.at[s]).wait()

        himask = jnp.full((16,), -0x10000, dtype=jnp.int32)

        def widen_lo(v):
            return plsc.bitcast(v << 16, jnp.float32)

        def widen_hi(v):
            return plsc.bitcast(v & himask, jnp.float32)

        def do_add(s):
            @plsc.parallel_loop(0, C, unroll=4)
            def _add_body(e):
                for g in range(Dw // 16):
                    sl = pl.ds(g * 16, 16)
                    va = bufa[s, e, sl]
                    vb = bufb[s, e, sl]
                    bufo[s, e, pl.ds(g * 32, 16)] = widen_lo(va) + widen_lo(vb)
                    bufo[s, e, pl.ds(g * 32 + 16, 16)] = widen_hi(va) + widen_hi(vb)

        fire_gather(0, 0)
        fire_gather(1, 1)

        def outer(jj, carry):
            for s in range(NBUF):
                j = jj * NBUF + s
                if s < 2:
                    @pl.when(jj >= 1)
                    def _w():
                        wait_write(j - 2, (s - 2) % NBUF)
                else:
                    wait_write(j - 2, s - 2)
                if s < NBUF - 2:
                    fire_gather(j + 2, (s + 2) % NBUF)
                else:
                    @pl.when(jj < nouter - 1)
                    def _g():
                        fire_gather(j + 2, (s + 2) % NBUF)
                wait_gather(j, s)
                do_add(s)
                fire_write(j, s)
            return carry

        lax.fori_loop(0, nouter, outer, 0, unroll=False)

        for t in (nchunks - 2, nchunks - 1):
            wait_write(t, t % NBUF)

    return sc_fn


def kernel(x, edge_index, W, b):
    n, d_in = x.shape
    d_out = W.shape[1]
    E = edge_index.shape[1]

    cols_a = np.array([32 * g + k for g in range(d_out // 32) for k in range(16)])
    perm_ab = np.concatenate([cols_a, cols_a + 16])
    b_ab = (0.5 * b)[perm_ab].reshape(1, d_out).astype(jnp.float32)
    y32 = _tc_matmul_pack(x, W[:, perm_ab], b_ab)

    sc_fn = _make_sc_gather_add(E, d_out, C=80, NBUF=5)
    return sc_fn(y32, edge_index)
